# Initial kernel scaffold; baseline (speedup 1.0000x reference)
import jax
import jax.numpy as jnp
from jax.experimental import pallas as pl


def kernel(xyz1, xyz2, points1, points2, W0, b0, g0, beta0, W1, b1, g1, beta1):
    raise NotImplementedError("write your pallas kernel here")


# speedup vs baseline: 17.4258x; 17.4258x over previous
import functools

import jax
import jax.numpy as jnp
from jax.experimental import pallas as pl
from jax.experimental.pallas import tpu as pltpu


def _knn_layer1_kernel(xyz1_ref, xyz2_ref, nb_ref, p1_ref, p2_ref, w0_ref,
                       b0_ref, x1_ref, stats_ref, *, S, bn):
    a = xyz1_ref[0]
    b = xyz2_ref[0]
    prod = jax.lax.dot_general(b.astype(jnp.bfloat16), a.astype(jnp.bfloat16),
                               (((0,), (0,)), ((), ())),
                               preferred_element_type=jnp.float32)
    na = jnp.sum(a * a, axis=0)
    dT = (-2.0 * prod + na[None, :]) + nb_ref[0]

    iota0 = jax.lax.broadcasted_iota(jnp.int32, (S, bn), 0)
    inf = jnp.float32(jnp.inf)
    fill = jnp.int32(S)

    m1 = jnp.min(dT, axis=0)
    a1 = jnp.min(jnp.where(dT == m1[None, :], iota0, fill), axis=0)
    d2 = jnp.where(iota0 == a1[None, :], inf, dT)
    m2 = jnp.min(d2, axis=0)
    a2 = jnp.min(jnp.where(d2 == m2[None, :], iota0, fill), axis=0)
    d3 = jnp.where(iota0 == a2[None, :], inf, d2)
    m3 = jnp.min(d3, axis=0)
    a3 = jnp.min(jnp.where(d3 == m3[None, :], iota0, fill), axis=0)

    r1 = 1.0 / (m1 + 1e-8)
    r2 = 1.0 / (m2 + 1e-8)
    r3 = 1.0 / (m3 + 1e-8)
    rn = r1 + r2 + r3
    w1 = r1 / rn
    w2 = r2 / rn
    w3 = r3 / rn

    wmat = (jnp.where(iota0 == a1[None, :], w1[None, :], 0.0)
            + jnp.where(iota0 == a2[None, :], w2[None, :], 0.0)
            + jnp.where(iota0 == a3[None, :], w3[None, :], 0.0))

    interp = jax.lax.dot_general(p2_ref[0], wmat, (((1,), (0,)), ((), ())),
                                 preferred_element_type=jnp.float32,
                                 precision=jax.lax.Precision.HIGHEST)
    feat = jnp.concatenate([p1_ref[0], interp], axis=0)
    x1 = jax.lax.dot_general(w0_ref[...].astype(jnp.bfloat16),
                             feat.astype(jnp.bfloat16),
                             (((1,), (0,)), ((), ())),
                             preferred_element_type=jnp.float32) + b0_ref[...]
    x1_ref[0] = x1

    st = jnp.concatenate([jnp.sum(x1, axis=1, keepdims=True),
                          jnp.sum(x1 * x1, axis=1, keepdims=True)], axis=1)

    @pl.when((pl.program_id(0) == 0) & (pl.program_id(1) == 0))
    def _init():
        stats_ref[...] = jnp.zeros_like(stats_ref)

    stats_ref[...] += st


def _bn_layer2_kernel(x1_ref, st1_ref, g0_ref, be0_ref, w1_ref, b1_ref,
                      x2_ref, st2_ref, *, count):
    st = st1_ref[...]
    mean = st[:, 0:1] * (1.0 / count)
    var = st[:, 1:2] * (1.0 / count) - mean * mean
    scale = jax.lax.rsqrt(var + 1e-5) * g0_ref[...]
    shift = be0_ref[...] - mean * scale
    h = jnp.maximum(x1_ref[0] * scale + shift, 0.0)
    x2 = jax.lax.dot_general(w1_ref[...].astype(jnp.bfloat16),
                             h.astype(jnp.bfloat16),
                             (((1,), (0,)), ((), ())),
                             preferred_element_type=jnp.float32) + b1_ref[...]
    x2_ref[0] = x2

    st2 = jnp.concatenate([jnp.sum(x2, axis=1, keepdims=True),
                           jnp.sum(x2 * x2, axis=1, keepdims=True)], axis=1)

    @pl.when((pl.program_id(0) == 0) & (pl.program_id(1) == 0))
    def _init():
        st2_ref[...] = jnp.zeros_like(st2_ref)

    st2_ref[...] += st2


def _bn_relu_kernel(x2_ref, st2_ref, g1_ref, be1_ref, out_ref, *, count):
    st = st2_ref[...]
    mean = st[:, 0:1] * (1.0 / count)
    var = st[:, 1:2] * (1.0 / count) - mean * mean
    scale = jax.lax.rsqrt(var + 1e-5) * g1_ref[...]
    shift = be1_ref[...] - mean * scale
    out_ref[0] = jnp.maximum(x2_ref[0] * scale + shift, 0.0)


def kernel(xyz1, xyz2, points1, points2, W0, b0, g0, beta0, W1, b1, g1, beta1):
    B, _, N = xyz1.shape
    S = xyz2.shape[2]
    D1 = points1.shape[1]
    D2 = points2.shape[1]
    C1 = W0.shape[0]
    C2 = W1.shape[0]
    count = float(B * N)

    b0c = b0.reshape(C1, 1)
    g0c = g0.reshape(C1, 1)
    be0c = beta0.reshape(C1, 1)
    b1c = b1.reshape(C2, 1)
    g1c = g1.reshape(C2, 1)
    be1c = beta1.reshape(C2, 1)

    nbcol = jnp.sum(xyz2.transpose(0, 2, 1) ** 2, -1)[:, :, None]

    bn = 512
    grid = (B, N // bn)
    params = pltpu.CompilerParams(dimension_semantics=("arbitrary", "arbitrary"))

    x1, st1 = pl.pallas_call(
        functools.partial(_knn_layer1_kernel, S=S, bn=bn),
        grid=grid,
        in_specs=[
            pl.BlockSpec((1, 3, bn), lambda b, n: (b, 0, n)),
            pl.BlockSpec((1, 3, S), lambda b, n: (b, 0, 0)),
            pl.BlockSpec((1, S, 1), lambda b, n: (b, 0, 0)),
            pl.BlockSpec((1, D1, bn), lambda b, n: (b, 0, n)),
            pl.BlockSpec((1, D2, S), lambda b, n: (b, 0, 0)),
            pl.BlockSpec((C1, D1 + D2), lambda b, n: (0, 0)),
            pl.BlockSpec((C1, 1), lambda b, n: (0, 0)),
        ],
        out_specs=[
            pl.BlockSpec((1, C1, bn), lambda b, n: (b, 0, n)),
            pl.BlockSpec((C1, 2), lambda b, n: (0, 0)),
        ],
        out_shape=[
            jax.ShapeDtypeStruct((B, C1, N), jnp.float32),
            jax.ShapeDtypeStruct((C1, 2), jnp.float32),
        ],
        compiler_params=params,
    )(xyz1, xyz2, nbcol, points1, points2, W0, b0c)

    x2, st2 = pl.pallas_call(
        functools.partial(_bn_layer2_kernel, count=count),
        grid=grid,
        in_specs=[
            pl.BlockSpec((1, C1, bn), lambda b, n: (b, 0, n)),
            pl.BlockSpec((C1, 2), lambda b, n: (0, 0)),
            pl.BlockSpec((C1, 1), lambda b, n: (0, 0)),
            pl.BlockSpec((C1, 1), lambda b, n: (0, 0)),
            pl.BlockSpec((C2, C1), lambda b, n: (0, 0)),
            pl.BlockSpec((C2, 1), lambda b, n: (0, 0)),
        ],
        out_specs=[
            pl.BlockSpec((1, C2, bn), lambda b, n: (b, 0, n)),
            pl.BlockSpec((C2, 2), lambda b, n: (0, 0)),
        ],
        out_shape=[
            jax.ShapeDtypeStruct((B, C2, N), jnp.float32),
            jax.ShapeDtypeStruct((C2, 2), jnp.float32),
        ],
        compiler_params=params,
    )(x1, st1, g0c, be0c, W1, b1c)

    out = pl.pallas_call(
        functools.partial(_bn_relu_kernel, count=count),
        grid=grid,
        in_specs=[
            pl.BlockSpec((1, C2, bn), lambda b, n: (b, 0, n)),
            pl.BlockSpec((C2, 2), lambda b, n: (0, 0)),
            pl.BlockSpec((C2, 1), lambda b, n: (0, 0)),
            pl.BlockSpec((C2, 1), lambda b, n: (0, 0)),
        ],
        out_specs=pl.BlockSpec((1, C2, bn), lambda b, n: (b, 0, n)),
        out_shape=jax.ShapeDtypeStruct((B, C2, N), jnp.float32),
        compiler_params=params,
    )(x2, st2, g1c, be1c)

    return out


# speedup vs baseline: 21.3005x; 1.2224x over previous
import functools

import jax
import jax.numpy as jnp
from jax.experimental import pallas as pl
from jax.experimental.pallas import tpu as pltpu


def _knn_layer1_kernel(xyz1_ref, xyz2_ref, nb_ref, p1_ref, p2_ref, w0_ref,
                       b0_ref, x1_ref, stats_ref, *, S, bn):
    a = xyz1_ref[0]
    b = xyz2_ref[0]
    prod = jax.lax.dot_general(b.astype(jnp.bfloat16), a.astype(jnp.bfloat16),
                               (((0,), (0,)), ((), ())),
                               preferred_element_type=jnp.float32)
    na = jnp.sum(a * a, axis=0)
    dT = (-2.0 * prod + na[None, :]) + nb_ref[0]

    iota0 = jax.lax.broadcasted_iota(jnp.int32, (S, bn), 0)
    inf = jnp.float32(jnp.inf)
    fill = jnp.int32(S)

    m1 = jnp.min(dT, axis=0)
    a1 = jnp.min(jnp.where(dT == m1[None, :], iota0, fill), axis=0)
    d2 = jnp.where(iota0 == a1[None, :], inf, dT)
    m2 = jnp.min(d2, axis=0)
    a2 = jnp.min(jnp.where(d2 == m2[None, :], iota0, fill), axis=0)
    d3 = jnp.where(iota0 == a2[None, :], inf, d2)
    m3 = jnp.min(d3, axis=0)
    a3 = jnp.min(jnp.where(d3 == m3[None, :], iota0, fill), axis=0)

    r1 = 1.0 / (m1 + 1e-8)
    r2 = 1.0 / (m2 + 1e-8)
    r3 = 1.0 / (m3 + 1e-8)
    rn = r1 + r2 + r3
    w1 = r1 / rn
    w2 = r2 / rn
    w3 = r3 / rn

    wmat = (jnp.where(iota0 == a1[None, :], w1[None, :], 0.0)
            + jnp.where(iota0 == a2[None, :], w2[None, :], 0.0)
            + jnp.where(iota0 == a3[None, :], w3[None, :], 0.0))

    p2 = p2_ref[0]
    dn = (((1,), (0,)), ((), ()))
    p2h = p2.astype(jnp.bfloat16)
    p2l = (p2 - p2h.astype(jnp.float32)).astype(jnp.bfloat16)
    wh = wmat.astype(jnp.bfloat16)
    wl = (wmat - wh.astype(jnp.float32)).astype(jnp.bfloat16)
    interp = (jax.lax.dot_general(p2h, wh, dn, preferred_element_type=jnp.float32)
              + jax.lax.dot_general(p2h, wl, dn, preferred_element_type=jnp.float32)
              + jax.lax.dot_general(p2l, wh, dn, preferred_element_type=jnp.float32))
    feat = jnp.concatenate([p1_ref[0], interp], axis=0)
    x1 = jax.lax.dot_general(w0_ref[...].astype(jnp.bfloat16),
                             feat.astype(jnp.bfloat16),
                             (((1,), (0,)), ((), ())),
                             preferred_element_type=jnp.float32) + b0_ref[...]
    x1_ref[0] = x1

    st = jnp.concatenate([jnp.sum(x1, axis=1, keepdims=True),
                          jnp.sum(x1 * x1, axis=1, keepdims=True)], axis=1)

    @pl.when((pl.program_id(0) == 0) & (pl.program_id(1) == 0))
    def _init():
        stats_ref[...] = jnp.zeros_like(stats_ref)

    stats_ref[...] += st


def _bn_layer2_kernel(x1_ref, st1_ref, g0_ref, be0_ref, w1_ref, b1_ref,
                      x2_ref, st2_ref, *, count):
    st = st1_ref[...]
    mean = st[:, 0:1] * (1.0 / count)
    var = st[:, 1:2] * (1.0 / count) - mean * mean
    scale = jax.lax.rsqrt(var + 1e-5) * g0_ref[...]
    shift = be0_ref[...] - mean * scale
    h = jnp.maximum(x1_ref[0] * scale + shift, 0.0)
    x2 = jax.lax.dot_general(w1_ref[...].astype(jnp.bfloat16),
                             h.astype(jnp.bfloat16),
                             (((1,), (0,)), ((), ())),
                             preferred_element_type=jnp.float32) + b1_ref[...]
    x2_ref[0] = x2

    st2 = jnp.concatenate([jnp.sum(x2, axis=1, keepdims=True),
                           jnp.sum(x2 * x2, axis=1, keepdims=True)], axis=1)

    @pl.when((pl.program_id(0) == 0) & (pl.program_id(1) == 0))
    def _init():
        st2_ref[...] = jnp.zeros_like(st2_ref)

    st2_ref[...] += st2


def _bn_relu_kernel(x2_ref, st2_ref, g1_ref, be1_ref, out_ref, *, count):
    st = st2_ref[...]
    mean = st[:, 0:1] * (1.0 / count)
    var = st[:, 1:2] * (1.0 / count) - mean * mean
    scale = jax.lax.rsqrt(var + 1e-5) * g1_ref[...]
    shift = be1_ref[...] - mean * scale
    out_ref[0] = jnp.maximum(x2_ref[0] * scale + shift, 0.0)


def kernel(xyz1, xyz2, points1, points2, W0, b0, g0, beta0, W1, b1, g1, beta1):
    B, _, N = xyz1.shape
    S = xyz2.shape[2]
    D1 = points1.shape[1]
    D2 = points2.shape[1]
    C1 = W0.shape[0]
    C2 = W1.shape[0]
    count = float(B * N)

    b0c = b0.reshape(C1, 1)
    g0c = g0.reshape(C1, 1)
    be0c = beta0.reshape(C1, 1)
    b1c = b1.reshape(C2, 1)
    g1c = g1.reshape(C2, 1)
    be1c = beta1.reshape(C2, 1)

    nbcol = jnp.sum(xyz2.transpose(0, 2, 1) ** 2, -1)[:, :, None]

    bn = 512
    grid = (B, N // bn)
    params = pltpu.CompilerParams(dimension_semantics=("arbitrary", "arbitrary"))

    x1, st1 = pl.pallas_call(
        functools.partial(_knn_layer1_kernel, S=S, bn=bn),
        grid=grid,
        in_specs=[
            pl.BlockSpec((1, 3, bn), lambda b, n: (b, 0, n)),
            pl.BlockSpec((1, 3, S), lambda b, n: (b, 0, 0)),
            pl.BlockSpec((1, S, 1), lambda b, n: (b, 0, 0)),
            pl.BlockSpec((1, D1, bn), lambda b, n: (b, 0, n)),
            pl.BlockSpec((1, D2, S), lambda b, n: (b, 0, 0)),
            pl.BlockSpec((C1, D1 + D2), lambda b, n: (0, 0)),
            pl.BlockSpec((C1, 1), lambda b, n: (0, 0)),
        ],
        out_specs=[
            pl.BlockSpec((1, C1, bn), lambda b, n: (b, 0, n)),
            pl.BlockSpec((C1, 2), lambda b, n: (0, 0)),
        ],
        out_shape=[
            jax.ShapeDtypeStruct((B, C1, N), jnp.float32),
            jax.ShapeDtypeStruct((C1, 2), jnp.float32),
        ],
        compiler_params=params,
    )(xyz1, xyz2, nbcol, points1, points2, W0, b0c)

    bn2 = 1024
    grid2 = (B, N // bn2)

    x2, st2 = pl.pallas_call(
        functools.partial(_bn_layer2_kernel, count=count),
        grid=grid2,
        in_specs=[
            pl.BlockSpec((1, C1, bn2), lambda b, n: (b, 0, n)),
            pl.BlockSpec((C1, 2), lambda b, n: (0, 0)),
            pl.BlockSpec((C1, 1), lambda b, n: (0, 0)),
            pl.BlockSpec((C1, 1), lambda b, n: (0, 0)),
            pl.BlockSpec((C2, C1), lambda b, n: (0, 0)),
            pl.BlockSpec((C2, 1), lambda b, n: (0, 0)),
        ],
        out_specs=[
            pl.BlockSpec((1, C2, bn2), lambda b, n: (b, 0, n)),
            pl.BlockSpec((C2, 2), lambda b, n: (0, 0)),
        ],
        out_shape=[
            jax.ShapeDtypeStruct((B, C2, N), jnp.float32),
            jax.ShapeDtypeStruct((C2, 2), jnp.float32),
        ],
        compiler_params=params,
    )(x1, st1, g0c, be0c, W1, b1c)

    out = pl.pallas_call(
        functools.partial(_bn_relu_kernel, count=count),
        grid=grid2,
        in_specs=[
            pl.BlockSpec((1, C2, bn2), lambda b, n: (b, 0, n)),
            pl.BlockSpec((C2, 2), lambda b, n: (0, 0)),
            pl.BlockSpec((C2, 1), lambda b, n: (0, 0)),
            pl.BlockSpec((C2, 1), lambda b, n: (0, 0)),
        ],
        out_specs=pl.BlockSpec((1, C2, bn2), lambda b, n: (b, 0, n)),
        out_shape=jax.ShapeDtypeStruct((B, C2, N), jnp.float32),
        compiler_params=params,
    )(x2, st2, g1c, be1c)

    return out


# speedup vs baseline: 22.0252x; 1.0340x over previous
import functools

import jax
import jax.numpy as jnp
from jax.experimental import pallas as pl
from jax.experimental.pallas import tpu as pltpu


def _knn_layer1_kernel(xyz1_ref, xyz2_ref, nb_ref, p1_ref, p2h_ref, p2l_ref,
                       w0_ref, b0_ref, x1_ref, stats_ref, *, S, bn):
    a = xyz1_ref[0]
    b = xyz2_ref[0]
    prod = jax.lax.dot_general(b.astype(jnp.bfloat16), a.astype(jnp.bfloat16),
                               (((0,), (0,)), ((), ())),
                               preferred_element_type=jnp.float32)
    na = jnp.sum(a * a, axis=0)
    dT = (-2.0 * prod + na[None, :]) + nb_ref[0]

    iota0 = jax.lax.broadcasted_iota(jnp.int32, (S, bn), 0)
    inf = jnp.float32(jnp.inf)
    fill = jnp.int32(S)

    m1 = jnp.min(dT, axis=0)
    a1 = jnp.min(jnp.where(dT == m1[None, :], iota0, fill), axis=0)
    d2 = jnp.where(iota0 == a1[None, :], inf, dT)
    m2 = jnp.min(d2, axis=0)
    a2 = jnp.min(jnp.where(d2 == m2[None, :], iota0, fill), axis=0)
    d3 = jnp.where(iota0 == a2[None, :], inf, d2)
    m3 = jnp.min(d3, axis=0)
    a3 = jnp.min(jnp.where(d3 == m3[None, :], iota0, fill), axis=0)

    r1 = 1.0 / (m1 + 1e-8)
    r2 = 1.0 / (m2 + 1e-8)
    r3 = 1.0 / (m3 + 1e-8)
    rn = r1 + r2 + r3
    w1 = r1 / rn
    w2 = r2 / rn
    w3 = r3 / rn

    wmat = (jnp.where(iota0 == a1[None, :], w1[None, :], 0.0)
            + jnp.where(iota0 == a2[None, :], w2[None, :], 0.0)
            + jnp.where(iota0 == a3[None, :], w3[None, :], 0.0))

    dn = (((1,), (0,)), ((), ()))
    wh = wmat.astype(jnp.bfloat16)
    interp = (jax.lax.dot_general(p2h_ref[0], wh, dn, preferred_element_type=jnp.float32)
              + jax.lax.dot_general(p2l_ref[0], wh, dn, preferred_element_type=jnp.float32))
    feat = jnp.concatenate([p1_ref[0], interp.astype(jnp.bfloat16)], axis=0)
    x1 = jax.lax.dot_general(w0_ref[...], feat,
                             (((1,), (0,)), ((), ())),
                             preferred_element_type=jnp.float32) + b0_ref[...]
    x1_ref[0] = x1

    st = jnp.concatenate([jnp.sum(x1, axis=1, keepdims=True),
                          jnp.sum(x1 * x1, axis=1, keepdims=True)], axis=1)

    @pl.when((pl.program_id(0) == 0) & (pl.program_id(1) == 0))
    def _init():
        stats_ref[...] = jnp.zeros_like(stats_ref)

    stats_ref[...] += st


def _bn_layer2_kernel(x1_ref, st1_ref, g0_ref, be0_ref, w1_ref, b1_ref,
                      x2_ref, st2_ref, *, count):
    st = st1_ref[...]
    mean = st[:, 0:1] * (1.0 / count)
    var = st[:, 1:2] * (1.0 / count) - mean * mean
    scale = jax.lax.rsqrt(var + 1e-5) * g0_ref[...]
    shift = be0_ref[...] - mean * scale
    h = jnp.maximum(x1_ref[0] * scale + shift, 0.0)
    x2 = jax.lax.dot_general(w1_ref[...], h.astype(jnp.bfloat16),
                             (((1,), (0,)), ((), ())),
                             preferred_element_type=jnp.float32) + b1_ref[...]
    x2_ref[0] = x2

    st2 = jnp.concatenate([jnp.sum(x2, axis=1, keepdims=True),
                           jnp.sum(x2 * x2, axis=1, keepdims=True)], axis=1)

    @pl.when((pl.program_id(0) == 0) & (pl.program_id(1) == 0))
    def _init():
        st2_ref[...] = jnp.zeros_like(st2_ref)

    st2_ref[...] += st2


def _bn_relu_kernel(x2_ref, st2_ref, g1_ref, be1_ref, out_ref, *, count):
    st = st2_ref[...]
    mean = st[:, 0:1] * (1.0 / count)
    var = st[:, 1:2] * (1.0 / count) - mean * mean
    scale = jax.lax.rsqrt(var + 1e-5) * g1_ref[...]
    shift = be1_ref[...] - mean * scale
    out_ref[0] = jnp.maximum(x2_ref[0] * scale + shift, 0.0)


def kernel(xyz1, xyz2, points1, points2, W0, b0, g0, beta0, W1, b1, g1, beta1):
    B, _, N = xyz1.shape
    S = xyz2.shape[2]
    D1 = points1.shape[1]
    D2 = points2.shape[1]
    C1 = W0.shape[0]
    C2 = W1.shape[0]
    count = float(B * N)

    b0c = b0.reshape(C1, 1)
    g0c = g0.reshape(C1, 1)
    be0c = beta0.reshape(C1, 1)
    b1c = b1.reshape(C2, 1)
    g1c = g1.reshape(C2, 1)
    be1c = beta1.reshape(C2, 1)

    nbcol = jnp.sum(xyz2.transpose(0, 2, 1) ** 2, -1)[:, :, None]
    w0bf = W0.astype(jnp.bfloat16)
    w1bf = W1.astype(jnp.bfloat16)
    p1bf = points1.astype(jnp.bfloat16)
    p2h = points2.astype(jnp.bfloat16)
    p2l = (points2 - p2h.astype(jnp.float32)).astype(jnp.bfloat16)

    bn = 512
    grid = (B, N // bn)
    params = pltpu.CompilerParams(dimension_semantics=("arbitrary", "arbitrary"))

    x1, st1 = pl.pallas_call(
        functools.partial(_knn_layer1_kernel, S=S, bn=bn),
        grid=grid,
        in_specs=[
            pl.BlockSpec((1, 3, bn), lambda b, n: (b, 0, n)),
            pl.BlockSpec((1, 3, S), lambda b, n: (b, 0, 0)),
            pl.BlockSpec((1, S, 1), lambda b, n: (b, 0, 0)),
            pl.BlockSpec((1, D1, bn), lambda b, n: (b, 0, n)),
            pl.BlockSpec((1, D2, S), lambda b, n: (b, 0, 0)),
            pl.BlockSpec((1, D2, S), lambda b, n: (b, 0, 0)),
            pl.BlockSpec((C1, D1 + D2), lambda b, n: (0, 0)),
            pl.BlockSpec((C1, 1), lambda b, n: (0, 0)),
        ],
        out_specs=[
            pl.BlockSpec((1, C1, bn), lambda b, n: (b, 0, n)),
            pl.BlockSpec((C1, 2), lambda b, n: (0, 0)),
        ],
        out_shape=[
            jax.ShapeDtypeStruct((B, C1, N), jnp.float32),
            jax.ShapeDtypeStruct((C1, 2), jnp.float32),
        ],
        compiler_params=params,
    )(xyz1, xyz2, nbcol, p1bf, p2h, p2l, w0bf, b0c)

    bn2 = 1024
    grid2 = (B, N // bn2)

    x2, st2 = pl.pallas_call(
        functools.partial(_bn_layer2_kernel, count=count),
        grid=grid2,
        in_specs=[
            pl.BlockSpec((1, C1, bn2), lambda b, n: (b, 0, n)),
            pl.BlockSpec((C1, 2), lambda b, n: (0, 0)),
            pl.BlockSpec((C1, 1), lambda b, n: (0, 0)),
            pl.BlockSpec((C1, 1), lambda b, n: (0, 0)),
            pl.BlockSpec((C2, C1), lambda b, n: (0, 0)),
            pl.BlockSpec((C2, 1), lambda b, n: (0, 0)),
        ],
        out_specs=[
            pl.BlockSpec((1, C2, bn2), lambda b, n: (b, 0, n)),
            pl.BlockSpec((C2, 2), lambda b, n: (0, 0)),
        ],
        out_shape=[
            jax.ShapeDtypeStruct((B, C2, N), jnp.float32),
            jax.ShapeDtypeStruct((C2, 2), jnp.float32),
        ],
        compiler_params=params,
    )(x1, st1, g0c, be0c, w1bf, b1c)

    out = pl.pallas_call(
        functools.partial(_bn_relu_kernel, count=count),
        grid=grid2,
        in_specs=[
            pl.BlockSpec((1, C2, bn2), lambda b, n: (b, 0, n)),
            pl.BlockSpec((C2, 2), lambda b, n: (0, 0)),
            pl.BlockSpec((C2, 1), lambda b, n: (0, 0)),
            pl.BlockSpec((C2, 1), lambda b, n: (0, 0)),
        ],
        out_specs=pl.BlockSpec((1, C2, bn2), lambda b, n: (b, 0, n)),
        out_shape=jax.ShapeDtypeStruct((B, C2, N), jnp.float32),
        compiler_params=params,
    )(x2, st2, g1c, be1c)

    return out


# speedup vs baseline: 25.9611x; 1.1787x over previous
import functools

import jax
import jax.numpy as jnp
from jax.experimental import pallas as pl
from jax.experimental.pallas import tpu as pltpu


def _knn_layer1_kernel(xyz1_ref, xyz2_ref, nb_ref, p1_ref, p2h_ref, p2l_ref,
                       w0_ref, b0_ref, x1_ref, stats_ref, *, S, bn):
    a = xyz1_ref[0]
    b = xyz2_ref[0]
    prod = jax.lax.dot_general(b.astype(jnp.bfloat16), a.astype(jnp.bfloat16),
                               (((0,), (0,)), ((), ())),
                               preferred_element_type=jnp.float32)
    na = jnp.sum(a * a, axis=0)
    dT = (-2.0 * prod + na[None, :]) + nb_ref[0]

    inf = jnp.float32(jnp.inf)
    m1 = jnp.min(dT, axis=0)
    mask1 = dT == m1[None, :]
    t2 = jnp.where(mask1, inf, dT)
    m2 = jnp.min(t2, axis=0)
    mask2 = t2 == m2[None, :]
    t3 = jnp.where(mask2, inf, t2)
    m3 = jnp.min(t3, axis=0)
    mask3 = t3 == m3[None, :]

    r1 = 1.0 / (m1 + 1e-8)
    r2 = 1.0 / (m2 + 1e-8)
    r3 = 1.0 / (m3 + 1e-8)
    rn = r1 + r2 + r3
    w1 = r1 / rn
    w2 = r2 / rn
    w3 = r3 / rn

    zero = jnp.zeros((), jnp.float32)
    wmat = jnp.where(mask1, w1[None, :],
                     jnp.where(mask2, w2[None, :],
                               jnp.where(mask3, w3[None, :], zero)))

    dn = (((1,), (0,)), ((), ()))
    wh = wmat.astype(jnp.bfloat16)
    wl = (wmat - wh.astype(jnp.float32)).astype(jnp.bfloat16)
    interp = (jax.lax.dot_general(p2h_ref[0], wh, dn, preferred_element_type=jnp.float32)
              + jax.lax.dot_general(p2l_ref[0], wh, dn, preferred_element_type=jnp.float32)
              + jax.lax.dot_general(p2h_ref[0], wl, dn, preferred_element_type=jnp.float32))
    feat = jnp.concatenate([p1_ref[0], interp.astype(jnp.bfloat16)], axis=0)
    x1 = jax.lax.dot_general(w0_ref[...], feat,
                             (((1,), (0,)), ((), ())),
                             preferred_element_type=jnp.float32) + b0_ref[...]
    x1_ref[0] = x1

    st = jnp.concatenate([jnp.sum(x1, axis=1, keepdims=True),
                          jnp.sum(x1 * x1, axis=1, keepdims=True)], axis=1)

    @pl.when((pl.program_id(0) == 0) & (pl.program_id(1) == 0))
    def _init():
        stats_ref[...] = jnp.zeros_like(stats_ref)

    stats_ref[...] += st


def _bn_layer2_kernel(x1_ref, st1_ref, g0_ref, be0_ref, w1_ref, b1_ref,
                      x2_ref, st2_ref, *, count):
    st = st1_ref[...]
    mean = st[:, 0:1] * (1.0 / count)
    var = st[:, 1:2] * (1.0 / count) - mean * mean
    scale = jax.lax.rsqrt(var + 1e-5) * g0_ref[...]
    shift = be0_ref[...] - mean * scale
    h = jnp.maximum(x1_ref[0] * scale + shift, 0.0)
    x2 = jax.lax.dot_general(w1_ref[...], h.astype(jnp.bfloat16),
                             (((1,), (0,)), ((), ())),
                             preferred_element_type=jnp.float32) + b1_ref[...]
    x2_ref[0] = x2

    st2 = jnp.concatenate([jnp.sum(x2, axis=1, keepdims=True),
                           jnp.sum(x2 * x2, axis=1, keepdims=True)], axis=1)

    @pl.when((pl.program_id(0) == 0) & (pl.program_id(1) == 0))
    def _init():
        st2_ref[...] = jnp.zeros_like(st2_ref)

    st2_ref[...] += st2


def _bn_relu_kernel(x2_ref, st2_ref, g1_ref, be1_ref, out_ref, *, count):
    st = st2_ref[...]
    mean = st[:, 0:1] * (1.0 / count)
    var = st[:, 1:2] * (1.0 / count) - mean * mean
    scale = jax.lax.rsqrt(var + 1e-5) * g1_ref[...]
    shift = be1_ref[...] - mean * scale
    out_ref[0] = jnp.maximum(x2_ref[0] * scale + shift, 0.0)


def kernel(xyz1, xyz2, points1, points2, W0, b0, g0, beta0, W1, b1, g1, beta1):
    B, _, N = xyz1.shape
    S = xyz2.shape[2]
    D1 = points1.shape[1]
    D2 = points2.shape[1]
    C1 = W0.shape[0]
    C2 = W1.shape[0]
    count = float(B * N)

    b0c = b0.reshape(C1, 1)
    g0c = g0.reshape(C1, 1)
    be0c = beta0.reshape(C1, 1)
    b1c = b1.reshape(C2, 1)
    g1c = g1.reshape(C2, 1)
    be1c = beta1.reshape(C2, 1)

    nbcol = jnp.sum(xyz2.transpose(0, 2, 1) ** 2, -1)[:, :, None]
    w0bf = W0.astype(jnp.bfloat16)
    w1bf = W1.astype(jnp.bfloat16)
    p1bf = points1.astype(jnp.bfloat16)
    p2h = points2.astype(jnp.bfloat16)
    p2l = (points2 - p2h.astype(jnp.float32)).astype(jnp.bfloat16)

    bn = 512
    grid = (B, N // bn)
    params = pltpu.CompilerParams(dimension_semantics=("arbitrary", "arbitrary"))

    x1, st1 = pl.pallas_call(
        functools.partial(_knn_layer1_kernel, S=S, bn=bn),
        grid=grid,
        in_specs=[
            pl.BlockSpec((1, 3, bn), lambda b, n: (b, 0, n)),
            pl.BlockSpec((1, 3, S), lambda b, n: (b, 0, 0)),
            pl.BlockSpec((1, S, 1), lambda b, n: (b, 0, 0)),
            pl.BlockSpec((1, D1, bn), lambda b, n: (b, 0, n)),
            pl.BlockSpec((1, D2, S), lambda b, n: (b, 0, 0)),
            pl.BlockSpec((1, D2, S), lambda b, n: (b, 0, 0)),
            pl.BlockSpec((C1, D1 + D2), lambda b, n: (0, 0)),
            pl.BlockSpec((C1, 1), lambda b, n: (0, 0)),
        ],
        out_specs=[
            pl.BlockSpec((1, C1, bn), lambda b, n: (b, 0, n)),
            pl.BlockSpec((C1, 2), lambda b, n: (0, 0)),
        ],
        out_shape=[
            jax.ShapeDtypeStruct((B, C1, N), jnp.float32),
            jax.ShapeDtypeStruct((C1, 2), jnp.float32),
        ],
        compiler_params=params,
    )(xyz1, xyz2, nbcol, p1bf, p2h, p2l, w0bf, b0c)

    bn2 = 1024
    grid2 = (B, N // bn2)

    x2, st2 = pl.pallas_call(
        functools.partial(_bn_layer2_kernel, count=count),
        grid=grid2,
        in_specs=[
            pl.BlockSpec((1, C1, bn2), lambda b, n: (b, 0, n)),
            pl.BlockSpec((C1, 2), lambda b, n: (0, 0)),
            pl.BlockSpec((C1, 1), lambda b, n: (0, 0)),
            pl.BlockSpec((C1, 1), lambda b, n: (0, 0)),
            pl.BlockSpec((C2, C1), lambda b, n: (0, 0)),
            pl.BlockSpec((C2, 1), lambda b, n: (0, 0)),
        ],
        out_specs=[
            pl.BlockSpec((1, C2, bn2), lambda b, n: (b, 0, n)),
            pl.BlockSpec((C2, 2), lambda b, n: (0, 0)),
        ],
        out_shape=[
            jax.ShapeDtypeStruct((B, C2, N), jnp.float32),
            jax.ShapeDtypeStruct((C2, 2), jnp.float32),
        ],
        compiler_params=params,
    )(x1, st1, g0c, be0c, w1bf, b1c)

    out = pl.pallas_call(
        functools.partial(_bn_relu_kernel, count=count),
        grid=grid2,
        in_specs=[
            pl.BlockSpec((1, C2, bn2), lambda b, n: (b, 0, n)),
            pl.BlockSpec((C2, 2), lambda b, n: (0, 0)),
            pl.BlockSpec((C2, 1), lambda b, n: (0, 0)),
            pl.BlockSpec((C2, 1), lambda b, n: (0, 0)),
        ],
        out_specs=pl.BlockSpec((1, C2, bn2), lambda b, n: (b, 0, n)),
        out_shape=jax.ShapeDtypeStruct((B, C2, N), jnp.float32),
        compiler_params=params,
    )(x2, st2, g1c, be1c)

    return out
